# baseline (speedup 1.0000x reference)
import functools

import jax
import jax.numpy as jnp
import numpy as np
from jax.experimental import pallas as pl
from jax.experimental.pallas import tpu as pltpu

B, NA, NC, NB, D, H, FF = 2, 256, 512, 256, 768, 12, 3072
N = NA + NC + NB
DH = D // H
ROWS = B * N
TILE = 256


def _ln_qkv_kernel(x_ref, g_ref, b_ref, w_ref, bq_ref, o_ref):
    x = x_ref[...]
    mu = jnp.mean(x, axis=-1, keepdims=True)
    var = jnp.mean((x - mu) ** 2, axis=-1, keepdims=True)
    xn = (x - mu) * jax.lax.rsqrt(var + 1e-5) * g_ref[0] + b_ref[0]
    acc = jax.lax.dot_general(
        xn.astype(jnp.bfloat16), w_ref[...],
        (((1,), (0,)), ((), ())), preferred_element_type=jnp.float32)
    o_ref[...] = acc + bq_ref[...]


def _group_of_tile(r):
    r4 = jax.lax.rem(r, 4)
    return (r4 >= 1).astype(jnp.int32) + (r4 >= 3).astype(jnp.int32)


def _ln_qkv(x_cat, g_stack, b_stack, wqkv16, bqkv):
    return pl.pallas_call(
        _ln_qkv_kernel,
        grid=(ROWS // TILE,),
        in_specs=[
            pl.BlockSpec((TILE, D), lambda r: (r, 0)),
            pl.BlockSpec((1, 1, D), lambda r: (_group_of_tile(r), 0, 0)),
            pl.BlockSpec((1, 1, D), lambda r: (_group_of_tile(r), 0, 0)),
            pl.BlockSpec((D, 3 * D), lambda r: (0, 0)),
            pl.BlockSpec((1, 3 * D), lambda r: (0, 0)),
        ],
        out_specs=pl.BlockSpec((TILE, 3 * D), lambda r: (r, 0)),
        out_shape=jax.ShapeDtypeStruct((ROWS, 3 * D), jnp.float32),
    )(x_cat, g_stack, b_stack, wqkv16, bqkv)


def _attn_kernel(q_ref, k_ref, v_ref, o_ref):
    q = q_ref[0, 0]
    k = k_ref[0, 0]
    v = v_ref[0, 0]
    s = jax.lax.dot_general(q, k, (((1,), (1,)), ((), ())),
                            preferred_element_type=jnp.float32)
    s = s * (1.0 / np.sqrt(DH))
    iq = jax.lax.broadcasted_iota(jnp.int32, (N, N), 0)
    ik = jax.lax.broadcasted_iota(jnp.int32, (N, N), 1)
    gq = (iq >= NA).astype(jnp.int32) + (iq >= NA + NC).astype(jnp.int32)
    gk = (ik >= NA).astype(jnp.int32) + (ik >= NA + NC).astype(jnp.int32)
    s = s + jnp.where(gk <= gq, 0.0, -1e9)
    m = jnp.max(s, axis=-1, keepdims=True)
    p = jnp.exp(s - m)
    p = p / jnp.sum(p, axis=-1, keepdims=True)
    o = jax.lax.dot_general(p.astype(jnp.bfloat16), v,
                            (((1,), (0,)), ((), ())),
                            preferred_element_type=jnp.float32)
    o_ref[0, 0] = o


def _attn(q, k, v):
    spec = pl.BlockSpec((1, 1, N, DH), lambda b, h: (b, h, 0, 0))
    return pl.pallas_call(
        _attn_kernel,
        grid=(B, H),
        in_specs=[spec, spec, spec],
        out_specs=spec,
        out_shape=jax.ShapeDtypeStruct((B, H, N, DH), jnp.float32),
    )(q, k, v)


def _oproj_kernel(o_ref, w_ref, b_ref, t_ref, y_ref):
    acc = jax.lax.dot_general(o_ref[...], w_ref[...],
                              (((1,), (0,)), ((), ())),
                              preferred_element_type=jnp.float32)
    y_ref[...] = acc + b_ref[...] + t_ref[...]


def _oproj(o16, wo16, bo, x_cat):
    return pl.pallas_call(
        _oproj_kernel,
        grid=(ROWS // TILE,),
        in_specs=[
            pl.BlockSpec((TILE, D), lambda r: (r, 0)),
            pl.BlockSpec((D, D), lambda r: (0, 0)),
            pl.BlockSpec((1, D), lambda r: (0, 0)),
            pl.BlockSpec((TILE, D), lambda r: (r, 0)),
        ],
        out_specs=pl.BlockSpec((TILE, D), lambda r: (r, 0)),
        out_shape=jax.ShapeDtypeStruct((ROWS, D), jnp.float32),
    )(o16, wo16, bo, x_cat)


def _gate_kernel(y_ref, g_ref, b_ref, wr_ref, xln_ref, w_ref, *, E, K):
    y = y_ref[...]
    mu = jnp.mean(y, axis=-1, keepdims=True)
    var = jnp.mean((y - mu) ** 2, axis=-1, keepdims=True)
    xn = (y - mu) * jax.lax.rsqrt(var + 1e-5) * g_ref[...] + b_ref[...]
    xln_ref[...] = xn
    logits = jax.lax.dot_general(xn, wr_ref[...], (((1,), (0,)), ((), ())),
                                 preferred_element_type=jnp.float32)
    m = jnp.max(logits, axis=-1, keepdims=True)
    p = jnp.exp(logits - m)
    p = p / jnp.sum(p, axis=-1, keepdims=True)
    lane = jax.lax.broadcasted_iota(jnp.int32, p.shape, 1)
    rem = p
    wsel = jnp.zeros_like(p)
    vsum = jnp.zeros_like(m)
    for _ in range(K):
        mx = jnp.max(rem, axis=-1, keepdims=True)
        idxs = jnp.where(rem == mx, lane, E)
        mi = jnp.min(idxs, axis=-1, keepdims=True)
        oh = lane == mi
        wsel = wsel + jnp.where(oh, p, 0.0)
        vsum = vsum + mx
        rem = jnp.where(oh, -1.0, rem)
    w_ref[...] = wsel / vsum


def _gate(y_g, g2, b2, wr, E, K):
    T2 = y_g.shape[0]
    return pl.pallas_call(
        functools.partial(_gate_kernel, E=E, K=K),
        grid=(1,),
        in_specs=[
            pl.BlockSpec((T2, D), lambda i: (0, 0)),
            pl.BlockSpec((1, D), lambda i: (0, 0)),
            pl.BlockSpec((1, D), lambda i: (0, 0)),
            pl.BlockSpec((D, E), lambda i: (0, 0)),
        ],
        out_specs=[
            pl.BlockSpec((T2, D), lambda i: (0, 0)),
            pl.BlockSpec((T2, E), lambda i: (0, 0)),
        ],
        out_shape=[
            jax.ShapeDtypeStruct((T2, D), jnp.float32),
            jax.ShapeDtypeStruct((T2, E), jnp.float32),
        ],
    )(y_g, g2, b2, wr)


def _moe_kernel(xln_ref, w_ref, w1_ref, b1_ref, w2_ref, b2_ref, y_ref,
                o_ref, *, E):
    e = pl.program_id(0)
    h = jax.lax.dot_general(xln_ref[...], w1_ref[0],
                            (((1,), (0,)), ((), ())),
                            preferred_element_type=jnp.float32)
    h = jax.nn.gelu(h + b1_ref[0])
    out = jax.lax.dot_general(h.astype(jnp.bfloat16), w2_ref[0],
                              (((1,), (0,)), ((), ())),
                              preferred_element_type=jnp.float32)
    out = out + b2_ref[0]
    lane = jax.lax.broadcasted_iota(jnp.int32, w_ref.shape, 1)
    col = jnp.sum(jnp.where(lane == e, w_ref[...], 0.0), axis=-1,
                  keepdims=True)

    @pl.when(e == 0)
    def _():
        o_ref[...] = y_ref[...]

    o_ref[...] = o_ref[...] + col * out


def _moe(xln16, w, w116, b1, w216, b2, y_g, E):
    T2 = xln16.shape[0]
    return pl.pallas_call(
        functools.partial(_moe_kernel, E=E),
        grid=(E,),
        in_specs=[
            pl.BlockSpec((T2, D), lambda e: (0, 0)),
            pl.BlockSpec((T2, E), lambda e: (0, 0)),
            pl.BlockSpec((1, D, FF), lambda e: (e, 0, 0)),
            pl.BlockSpec((1, 1, FF), lambda e: (e, 0, 0)),
            pl.BlockSpec((1, FF, D), lambda e: (e, 0, 0)),
            pl.BlockSpec((1, 1, D), lambda e: (e, 0, 0)),
            pl.BlockSpec((T2, D), lambda e: (0, 0)),
        ],
        out_specs=pl.BlockSpec((T2, D), lambda e: (0, 0)),
        out_shape=jax.ShapeDtypeStruct((T2, D), jnp.float32),
    )(xln16, w, w116, b1, w216, b2, y_g)


def _moe_block(y_g, g2, b2v, wr, w116, b1, w216, b2, E, K):
    xln, w = _gate(y_g, g2, b2v, wr, E, K)
    return _moe(xln.astype(jnp.bfloat16), w, w116, b1, w216, b2, y_g, E)


def kernel(tokens_A, tokens_B, tokens_C, Wqkv, bqkv, Wo, bo,
           Wr_A, W1_A, b1_A, W2_A, b2_A, ln1g_A, ln1b_A, ln2g_A, ln2b_A,
           Wr_B, W1_B, b1_B, W2_B, b2_B, ln1g_B, ln1b_B, ln2g_B, ln2b_B,
           Wr_C, W1_C, b1_C, W2_C, b2_C, ln1g_C, ln1b_C, ln2g_C, ln2b_C):
    x_cat = jnp.concatenate([tokens_A, tokens_C, tokens_B],
                            axis=1).reshape(ROWS, D)
    g_stack = jnp.stack([ln1g_A, ln1g_C, ln1g_B]).reshape(3, 1, D)
    b_stack = jnp.stack([ln1b_A, ln1b_C, ln1b_B]).reshape(3, 1, D)

    qkv = _ln_qkv(x_cat, g_stack, b_stack, Wqkv.astype(jnp.bfloat16),
                  bqkv.reshape(1, 3 * D))
    qkv = qkv.reshape(B, N, 3, H, DH).transpose(2, 0, 3, 1, 4)
    qkv16 = qkv.astype(jnp.bfloat16)
    o = _attn(qkv16[0], qkv16[1], qkv16[2])
    o16 = o.transpose(0, 2, 1, 3).reshape(ROWS, D).astype(jnp.bfloat16)
    y_cat = _oproj(o16, Wo.astype(jnp.bfloat16), bo.reshape(1, D), x_cat)

    y = y_cat.reshape(B, N, D)
    yA = y[:, :NA].reshape(B * NA, D)
    yC = y[:, NA:NA + NC].reshape(B * NC, D)
    yB = y[:, NA + NC:].reshape(B * NB, D)

    def prep(W1, b1, W2, b2):
        E = W1.shape[0]
        return (W1.astype(jnp.bfloat16), b1.reshape(E, 1, FF),
                W2.astype(jnp.bfloat16), b2.reshape(E, 1, D))

    w1a, b1a, w2a, b2a = prep(W1_A, b1_A, W2_A, b2_A)
    w1b, b1b, w2b, b2b = prep(W1_B, b1_B, W2_B, b2_B)
    w1c, b1c, w2c, b2c = prep(W1_C, b1_C, W2_C, b2_C)

    outA = _moe_block(yA, ln2g_A.reshape(1, D), ln2b_A.reshape(1, D),
                      Wr_A, w1a, b1a, w2a, b2a, 4, 2)
    outC = _moe_block(yC, ln2g_C.reshape(1, D), ln2b_C.reshape(1, D),
                      Wr_C, w1c, b1c, w2c, b2c, 6, 1)
    outB = _moe_block(yB, ln2g_B.reshape(1, D), ln2b_B.reshape(1, D),
                      Wr_B, w1b, b1b, w2b, b2b, 4, 2)

    return (outA.reshape(B, NA, D), outB.reshape(B, NB, D),
            outC.reshape(B, NC, D))


# speedup vs baseline: 1.0135x; 1.0135x over previous
import functools

import jax
import jax.numpy as jnp
import numpy as np
from jax.experimental import pallas as pl
from jax.experimental.pallas import tpu as pltpu

B, NA, NC, NB, D, H, FF = 2, 256, 512, 256, 768, 12, 3072
N = NA + NC + NB
TILE_S = 128


def _lnorm(x, g, b):
    mu = jnp.mean(x, axis=-1, keepdims=True)
    var = jnp.var(x, axis=-1, keepdims=True)
    return (x - mu) / jnp.sqrt(var + 1e-5) * g + b


def _mask_add_np():
    gid = np.concatenate([np.zeros(NA), np.ones(NC), 2 * np.ones(NB)])
    allowed = gid[None, :] <= gid[:, None]
    return jnp.asarray(np.where(allowed, 0.0, -1e9), dtype=jnp.float32)


def _mha(x, Wqkv, bqkv, Wo, bo, mask_add):
    Bq, Nq, Dm = x.shape
    qkv = x @ Wqkv + bqkv
    q, k, v = jnp.split(qkv, 3, axis=-1)
    dh = Dm // H

    def r(t):
        return t.reshape(Bq, Nq, H, dh).transpose(0, 2, 1, 3)

    q, k, v = r(q), r(k), r(v)
    att = jnp.einsum('bhqd,bhkd->bhqk', q, k) / np.sqrt(dh)
    att = jax.nn.softmax(att + mask_add[None, None], axis=-1)
    o = jnp.einsum('bhqk,bhkd->bhqd', att, v)
    return o.transpose(0, 2, 1, 3).reshape(Bq, Nq, Dm) @ Wo + bo


def _routing_tables(idx, val, T2, E, K, n_tiles):
    S = T2 * K
    e_flat = idx.reshape(S)
    g_flat = val.reshape(S)
    t_flat = (jnp.arange(S, dtype=jnp.int32) // K)
    oh = (e_flat[:, None] == jnp.arange(E, dtype=jnp.int32)[None, :])
    oh = oh.astype(jnp.int32)
    pref = jnp.cumsum(oh, axis=0)
    rank = jnp.take_along_axis(pref, e_flat[:, None], axis=1)[:, 0] - 1
    counts = pref[-1]
    tiles_e = (counts + TILE_S - 1) // TILE_S
    tile_start = jnp.concatenate([jnp.zeros(1, jnp.int32),
                                  jnp.cumsum(tiles_e)[:-1].astype(jnp.int32)])
    row_start = tile_start * TILE_S
    pos = row_start[e_flat] + rank
    S_pad = n_tiles * TILE_S
    row_token = jnp.full((S_pad,), T2, jnp.int32).at[pos].set(t_flat)
    row_gate = jnp.zeros((S_pad,), jnp.float32).at[pos].set(g_flat)
    total_tiles = jnp.sum(tiles_e)
    ar = jnp.arange(n_tiles, dtype=jnp.int32)
    tile_expert = jnp.searchsorted(jnp.cumsum(tiles_e), ar,
                                   side='right').astype(jnp.int32)
    tile_expert = jnp.clip(tile_expert, 0, E - 1)
    tile_valid = (ar < total_tiles).astype(jnp.int32)
    return (row_token.reshape(S_pad, 1), row_gate.reshape(S_pad, 1),
            tile_expert, tile_valid)


def _smoe_kernel(te_ref, tv_ref, x_ref, tok_ref, gate_ref, w1_ref, b1_ref,
                 w2_ref, b2_ref, y_ref, o_ref, *, T2):
    i = pl.program_id(0)

    @pl.when(i == 0)
    def _():
        o_ref[...] = y_ref[...]

    @pl.when(tv_ref[i] == 1)
    def _():
        tok = tok_ref[...]
        colt = jax.lax.broadcasted_iota(jnp.int32, (TILE_S, T2), 1)
        P = (colt == tok).astype(jnp.bfloat16)
        xs = jax.lax.dot_general(P, x_ref[...], (((1,), (0,)), ((), ())),
                                 preferred_element_type=jnp.float32)
        h = jax.lax.dot_general(xs.astype(jnp.bfloat16), w1_ref[0],
                                (((1,), (0,)), ((), ())),
                                preferred_element_type=jnp.float32)
        h = jax.nn.gelu(h + b1_ref[0])
        out = jax.lax.dot_general(h.astype(jnp.bfloat16), w2_ref[0],
                                  (((1,), (0,)), ((), ())),
                                  preferred_element_type=jnp.float32)
        out = (out + b2_ref[0]) * gate_ref[...]
        o_ref[...] = o_ref[...] + jax.lax.dot_general(
            P, out.astype(jnp.bfloat16), (((0,), (0,)), ((), ())),
            preferred_element_type=jnp.float32)


def _smoe(xln16, row_token, row_gate, tile_expert, tile_valid,
          w116, b1, w216, b2, y_g, E, n_tiles):
    T2 = xln16.shape[0]
    grid_spec = pltpu.PrefetchScalarGridSpec(
        num_scalar_prefetch=2,
        grid=(n_tiles,),
        in_specs=[
            pl.BlockSpec((T2, D), lambda i, te, tv: (0, 0)),
            pl.BlockSpec((TILE_S, 1), lambda i, te, tv: (i, 0)),
            pl.BlockSpec((TILE_S, 1), lambda i, te, tv: (i, 0)),
            pl.BlockSpec((1, D, FF), lambda i, te, tv: (te[i], 0, 0)),
            pl.BlockSpec((1, 1, FF), lambda i, te, tv: (te[i], 0, 0)),
            pl.BlockSpec((1, FF, D), lambda i, te, tv: (te[i], 0, 0)),
            pl.BlockSpec((1, 1, D), lambda i, te, tv: (te[i], 0, 0)),
            pl.BlockSpec((T2, D), lambda i, te, tv: (0, 0)),
        ],
        out_specs=pl.BlockSpec((T2, D), lambda i, te, tv: (0, 0)),
    )
    return pl.pallas_call(
        functools.partial(_smoe_kernel, T2=T2),
        grid_spec=grid_spec,
        out_shape=jax.ShapeDtypeStruct((T2, D), jnp.float32),
    )(tile_expert, tile_valid, xln16, row_token, row_gate,
      w116, b1, w216, b2, y_g)


def _moe_block(t_g, Wr, W1, b1, W2, b2, g2, b2v, E, K):
    Bq, T, _ = t_g.shape
    T2 = Bq * T
    x = _lnorm(t_g, g2, b2v)
    logits = jnp.einsum('btd,de->bte', x, Wr)
    probs = jax.nn.softmax(logits, axis=-1)
    vals, sidx = jax.lax.top_k(probs, K)
    vals = vals / jnp.sum(vals, axis=-1, keepdims=True)

    n_tiles = (T2 * K) // TILE_S + E
    idx = sidx.reshape(T2, K).astype(jnp.int32)
    val = vals.reshape(T2, K)
    row_token, row_gate, tile_expert, tile_valid = _routing_tables(
        idx, val, T2, E, K, n_tiles)
    out = _smoe(x.reshape(T2, D).astype(jnp.bfloat16), row_token, row_gate,
                tile_expert, tile_valid,
                W1.astype(jnp.bfloat16), b1.reshape(E, 1, FF),
                W2.astype(jnp.bfloat16), b2.reshape(E, 1, D),
                t_g.reshape(T2, D), E, n_tiles)
    return out.reshape(Bq, T, D)


def kernel(tokens_A, tokens_B, tokens_C, Wqkv, bqkv, Wo, bo,
           Wr_A, W1_A, b1_A, W2_A, b2_A, ln1g_A, ln1b_A, ln2g_A, ln2b_A,
           Wr_B, W1_B, b1_B, W2_B, b2_B, ln1g_B, ln1b_B, ln2g_B, ln2b_B,
           Wr_C, W1_C, b1_C, W2_C, b2_C, ln1g_C, ln1b_C, ln2g_C, ln2b_C):
    m = _mask_add_np()
    x = jnp.concatenate([
        _lnorm(tokens_A, ln1g_A, ln1b_A),
        _lnorm(tokens_C, ln1g_C, ln1b_C),
        _lnorm(tokens_B, ln1g_B, ln1b_B),
    ], axis=1)
    attn = _mha(x, Wqkv, bqkv, Wo, bo, m)
    tA = tokens_A + attn[:, :NA]
    tC = tokens_C + attn[:, NA:NA + NC]
    tB = tokens_B + attn[:, NA + NC:]

    outA = _moe_block(tA, Wr_A, W1_A, b1_A, W2_A, b2_A, ln2g_A, ln2b_A, 4, 2)
    outC = _moe_block(tC, Wr_C, W1_C, b1_C, W2_C, b2_C, ln2g_C, ln2b_C, 6, 1)
    outB = _moe_block(tB, Wr_B, W1_B, b1_B, W2_B, b2_B, ln2g_B, ln2b_B, 4, 2)

    return outA, outB, outC


# speedup vs baseline: 1.1527x; 1.1374x over previous
import functools

import jax
import jax.numpy as jnp
import numpy as np
from jax.experimental import pallas as pl
from jax.experimental.pallas import tpu as pltpu

B, NA, NC, NB, D, H, FF = 2, 256, 512, 256, 768, 12, 3072
N = NA + NC + NB
TILE_S = 128


def _lnorm(x, g, b):
    mu = jnp.mean(x, axis=-1, keepdims=True)
    var = jnp.var(x, axis=-1, keepdims=True)
    return (x - mu) / jnp.sqrt(var + 1e-5) * g + b


def _mask_add_np():
    gid = np.concatenate([np.zeros(NA), np.ones(NC), 2 * np.ones(NB)])
    allowed = gid[None, :] <= gid[:, None]
    return jnp.asarray(np.where(allowed, 0.0, -1e9), dtype=jnp.float32)


def _mha(x, Wqkv, bqkv, Wo, bo, mask_add):
    Bq, Nq, Dm = x.shape
    qkv = x @ Wqkv + bqkv
    q, k, v = jnp.split(qkv, 3, axis=-1)
    dh = Dm // H

    def r(t):
        return t.reshape(Bq, Nq, H, dh).transpose(0, 2, 1, 3)

    q, k, v = r(q), r(k), r(v)
    att = jnp.einsum('bhqd,bhkd->bhqk', q, k) / np.sqrt(dh)
    att = jax.nn.softmax(att + mask_add[None, None], axis=-1)
    o = jnp.einsum('bhqk,bhkd->bhqd', att, v)
    return o.transpose(0, 2, 1, 3).reshape(Bq, Nq, Dm) @ Wo + bo


EPAD = 8


def _route_kernel(idx_ref, val_ref, tok_ref, gate_ref, te_ref, tv_ref,
                  *, S, S_pad, T2, E, K, n_tiles):
    e = idx_ref[...]
    g = val_ref[...]
    lane8 = jax.lax.broadcasted_iota(jnp.int32, (S, EPAD), 1)
    oh = (lane8 == e).astype(jnp.bfloat16)
    ri = jax.lax.broadcasted_iota(jnp.int32, (S, S), 0)
    ci = jax.lax.broadcasted_iota(jnp.int32, (S, S), 1)
    ltri = (ci < ri).astype(jnp.bfloat16)
    rank = jax.lax.dot_general(ltri, oh, (((1,), (0,)), ((), ())),
                               preferred_element_type=jnp.float32)
    ohf = oh.astype(jnp.float32)
    counts_row = jnp.sum(ohf, axis=0, keepdims=True)
    tiles_row = jnp.floor((counts_row + (TILE_S - 1)) * (1.0 / TILE_S))
    er = jax.lax.broadcasted_iota(jnp.int32, (EPAD, EPAD), 0)
    ec = jax.lax.broadcasted_iota(jnp.int32, (EPAD, EPAD), 1)
    sltri8 = (er < ec).astype(jnp.bfloat16)
    start_row = jax.lax.dot_general(tiles_row.astype(jnp.bfloat16), sltri8,
                                    (((1,), (0,)), ((), ())),
                                    preferred_element_type=jnp.float32)
    pos = jnp.sum(ohf * (start_row * TILE_S + rank), axis=1,
                  keepdims=True)
    rr = jax.lax.broadcasted_iota(jnp.int32, (S, S_pad), 1)
    m2 = (rr == pos.astype(jnp.int32)).astype(jnp.bfloat16)
    tshift = jax.lax.broadcasted_iota(jnp.int32, (S, 1), 0) // K - T2
    rhs = jnp.concatenate([tshift.astype(jnp.float32), g], axis=1)
    inv = jax.lax.dot_general(m2.astype(jnp.float32), rhs,
                              (((0,), (0,)), ((), ())),
                              preferred_element_type=jnp.float32,
                              precision=jax.lax.Precision.HIGHEST)
    tok_ref[...] = (inv[:, 0:1] + T2).astype(jnp.int32)
    gate_ref[...] = inv[:, 1:2]
    incl8 = (er <= ec).astype(jnp.bfloat16)
    cum_col = jax.lax.dot_general(incl8, tiles_row.astype(jnp.bfloat16),
                                  (((0,), (1,)), ((), ())),
                                  preferred_element_type=jnp.float32)
    ti = jax.lax.broadcasted_iota(jnp.int32, (EPAD, n_tiles), 1)
    te = jnp.sum((cum_col.astype(jnp.int32) <= ti).astype(jnp.float32),
                 axis=0, keepdims=True)
    te_ref[...] = jnp.clip(te, 0, E - 1).astype(jnp.int32)
    total = jnp.sum(tiles_row, axis=1, keepdims=True)
    ti1 = jax.lax.broadcasted_iota(jnp.int32, (1, n_tiles), 1)
    tv_ref[...] = (ti1 < total.astype(jnp.int32)).astype(jnp.int32)


def _routing_tables(idx, val, T2, E, K, n_tiles):
    S = T2 * K
    S_pad = n_tiles * TILE_S
    row_token, row_gate, te, tv = pl.pallas_call(
        functools.partial(_route_kernel, S=S, S_pad=S_pad, T2=T2, E=E, K=K,
                          n_tiles=n_tiles),
        grid=(1,),
        in_specs=[
            pl.BlockSpec((S, 1), lambda i: (0, 0)),
            pl.BlockSpec((S, 1), lambda i: (0, 0)),
        ],
        out_specs=[
            pl.BlockSpec((S_pad, 1), lambda i: (0, 0)),
            pl.BlockSpec((S_pad, 1), lambda i: (0, 0)),
            pl.BlockSpec((1, n_tiles), lambda i: (0, 0)),
            pl.BlockSpec((1, n_tiles), lambda i: (0, 0)),
        ],
        out_shape=[
            jax.ShapeDtypeStruct((S_pad, 1), jnp.int32),
            jax.ShapeDtypeStruct((S_pad, 1), jnp.float32),
            jax.ShapeDtypeStruct((1, n_tiles), jnp.int32),
            jax.ShapeDtypeStruct((1, n_tiles), jnp.int32),
        ],
    )(idx.reshape(S, 1), val.reshape(S, 1))
    return row_token, row_gate, te.reshape(n_tiles), tv.reshape(n_tiles)


def _smoe_kernel(te_ref, tv_ref, x_ref, tok_ref, gate_ref, w1_ref, b1_ref,
                 w2_ref, b2_ref, y_ref, o_ref, *, T2):
    i = pl.program_id(0)

    @pl.when(i == 0)
    def _():
        o_ref[...] = y_ref[...]

    @pl.when(tv_ref[i] == 1)
    def _():
        tok = tok_ref[...]
        colt = jax.lax.broadcasted_iota(jnp.int32, (TILE_S, T2), 1)
        P = (colt == tok).astype(jnp.bfloat16)
        xs = jax.lax.dot_general(P, x_ref[...], (((1,), (0,)), ((), ())),
                                 preferred_element_type=jnp.float32)
        h = jax.lax.dot_general(xs.astype(jnp.bfloat16), w1_ref[0],
                                (((1,), (0,)), ((), ())),
                                preferred_element_type=jnp.float32)
        h = jax.nn.gelu(h + b1_ref[0])
        out = jax.lax.dot_general(h.astype(jnp.bfloat16), w2_ref[0],
                                  (((1,), (0,)), ((), ())),
                                  preferred_element_type=jnp.float32)
        out = (out + b2_ref[0]) * gate_ref[...]
        o_ref[...] = o_ref[...] + jax.lax.dot_general(
            P, out.astype(jnp.bfloat16), (((0,), (0,)), ((), ())),
            preferred_element_type=jnp.float32)


def _smoe(xln16, row_token, row_gate, tile_expert, tile_valid,
          w116, b1, w216, b2, y_g, E, n_tiles):
    T2 = xln16.shape[0]
    grid_spec = pltpu.PrefetchScalarGridSpec(
        num_scalar_prefetch=2,
        grid=(n_tiles,),
        in_specs=[
            pl.BlockSpec((T2, D), lambda i, te, tv: (0, 0)),
            pl.BlockSpec((TILE_S, 1), lambda i, te, tv: (i, 0)),
            pl.BlockSpec((TILE_S, 1), lambda i, te, tv: (i, 0)),
            pl.BlockSpec((1, D, FF), lambda i, te, tv: (te[i], 0, 0)),
            pl.BlockSpec((1, 1, FF), lambda i, te, tv: (te[i], 0, 0)),
            pl.BlockSpec((1, FF, D), lambda i, te, tv: (te[i], 0, 0)),
            pl.BlockSpec((1, 1, D), lambda i, te, tv: (te[i], 0, 0)),
            pl.BlockSpec((T2, D), lambda i, te, tv: (0, 0)),
        ],
        out_specs=pl.BlockSpec((T2, D), lambda i, te, tv: (0, 0)),
    )
    return pl.pallas_call(
        functools.partial(_smoe_kernel, T2=T2),
        grid_spec=grid_spec,
        out_shape=jax.ShapeDtypeStruct((T2, D), jnp.float32),
    )(tile_expert, tile_valid, xln16, row_token, row_gate,
      w116, b1, w216, b2, y_g)


def _moe_block(t_g, Wr, W1, b1, W2, b2, g2, b2v, E, K):
    Bq, T, _ = t_g.shape
    T2 = Bq * T
    x = _lnorm(t_g, g2, b2v)
    logits = jnp.einsum('btd,de->bte', x, Wr)
    probs = jax.nn.softmax(logits, axis=-1)
    vals, sidx = jax.lax.top_k(probs, K)
    vals = vals / jnp.sum(vals, axis=-1, keepdims=True)

    n_tiles = (T2 * K) // TILE_S + E
    idx = sidx.reshape(T2, K).astype(jnp.int32)
    val = vals.reshape(T2, K)
    row_token, row_gate, tile_expert, tile_valid = _routing_tables(
        idx, val, T2, E, K, n_tiles)
    out = _smoe(x.reshape(T2, D).astype(jnp.bfloat16), row_token, row_gate,
                tile_expert, tile_valid,
                W1.astype(jnp.bfloat16), b1.reshape(E, 1, FF),
                W2.astype(jnp.bfloat16), b2.reshape(E, 1, D),
                t_g.reshape(T2, D), E, n_tiles)
    return out.reshape(Bq, T, D)


def kernel(tokens_A, tokens_B, tokens_C, Wqkv, bqkv, Wo, bo,
           Wr_A, W1_A, b1_A, W2_A, b2_A, ln1g_A, ln1b_A, ln2g_A, ln2b_A,
           Wr_B, W1_B, b1_B, W2_B, b2_B, ln1g_B, ln1b_B, ln2g_B, ln2b_B,
           Wr_C, W1_C, b1_C, W2_C, b2_C, ln1g_C, ln1b_C, ln2g_C, ln2b_C):
    m = _mask_add_np()
    x = jnp.concatenate([
        _lnorm(tokens_A, ln1g_A, ln1b_A),
        _lnorm(tokens_C, ln1g_C, ln1b_C),
        _lnorm(tokens_B, ln1g_B, ln1b_B),
    ], axis=1)
    attn = _mha(x, Wqkv, bqkv, Wo, bo, m)
    tA = tokens_A + attn[:, :NA]
    tC = tokens_C + attn[:, NA:NA + NC]
    tB = tokens_B + attn[:, NA + NC:]

    outA = _moe_block(tA, Wr_A, W1_A, b1_A, W2_A, b2_A, ln2g_A, ln2b_A, 4, 2)
    outC = _moe_block(tC, Wr_C, W1_C, b1_C, W2_C, b2_C, ln2g_C, ln2b_C, 6, 1)
    outB = _moe_block(tB, Wr_B, W1_B, b1_B, W2_B, b2_B, ln2g_B, ln2b_B, 4, 2)

    return outA, outB, outC
